# baseline (speedup 1.0000x reference)
import functools

import jax
import jax.numpy as jnp
from jax import lax
from jax.experimental import pallas as pl
from jax.experimental.pallas import tpu as pltpu
from jax.experimental.pallas import tpu_sc as plsc

B, L, D = 1024, 50, 512
F = L + 2
_info = plsc.get_sparse_core_info()
_NC, _NS = _info.num_cores, _info.num_subcores
_NW = _NC * _NS
_BPW = B // _NW


@functools.partial(
    pl.kernel,
    out_type=jax.ShapeDtypeStruct((B, F, D), jnp.float32),
    mesh=plsc.VectorSubcoreMesh(core_axis_name="c", subcore_axis_name="s"),
    scratch_types=[
        pltpu.VMEM((_BPW,), jnp.int32),
        pltpu.VMEM((_BPW,), jnp.int32),
        pltpu.VMEM((_BPW, D), jnp.float32),
        pltpu.VMEM((_BPW, D), jnp.float32),
        pltpu.VMEM((2, L, D), jnp.float32),
        pltpu.VMEM((F, D), jnp.float32),
        pltpu.SemaphoreType.DMA,
        pltpu.SemaphoreType.DMA,
        pltpu.SemaphoreType.DMA((2,)),
        pltpu.SemaphoreType.DMA,
    ],
)
def _sc_all(f_hbm, gidx_hbm, didx_hbm, wg_hbm, wd_hbm, out_hbm,
            gidx_v, didx_v, grows_v, drows_v, fbuf, obuf,
            sem_g, sem_d, sem_in, sem_out):
    wid = lax.axis_index("s") * _NC + lax.axis_index("c")
    base = wid * _BPW
    pltpu.sync_copy(gidx_hbm.at[pl.ds(base, _BPW)], gidx_v)
    pltpu.sync_copy(didx_hbm.at[pl.ds(base, _BPW)], didx_v)
    cg = pltpu.async_copy(wg_hbm.at[gidx_v], grows_v, sem_g)
    cd = pltpu.async_copy(wd_hbm.at[didx_v], drows_v, sem_d)

    def in_copy(k):
        r = lax.rem(k, 2)
        return pltpu.make_async_copy(f_hbm.at[base + k], fbuf.at[r],
                                     sem_in.at[r])

    def out_copy(k):
        return pltpu.make_async_copy(obuf, out_hbm.at[base + k], sem_out)

    in_copy(0).start()
    in_copy(1).start()
    cg.wait()
    cd.wait()

    def step(k, _):
        r = lax.rem(k, 2)
        in_copy(k).wait()

        def shift_row(j, _):
            for i in range(D // 16):
                sl = pl.ds(i * 16, 16)
                obuf[j + 2, sl] = fbuf[r, j, sl]
            return 0

        lax.fori_loop(0, L, shift_row, 0)
        for i in range(D // 16):
            sl = pl.ds(i * 16, 16)
            obuf[0, sl] = grows_v[k, sl]
            obuf[1, sl] = drows_v[k, sl]
        out_copy(k).start()

        @pl.when(k + 2 < _BPW)
        def _():
            in_copy(k + 2).start()

        out_copy(k).wait()
        return 0

    lax.fori_loop(0, _BPW, step, 0)


def kernel(feature, genre_index, difficulty_index, W_genre, W_difficulty):
    gidx = genre_index.reshape(B).astype(jnp.int32)
    didx = difficulty_index.reshape(B).astype(jnp.int32)
    return _sc_all(feature, gidx, didx, W_genre, W_difficulty)


# speedup vs baseline: 1.0682x; 1.0682x over previous
import functools

import jax
import jax.numpy as jnp
from jax import lax
from jax.experimental import pallas as pl
from jax.experimental.pallas import tpu as pltpu
from jax.experimental.pallas import tpu_sc as plsc

B, L, D = 1024, 50, 512
F = L + 2
G = 8
_info = plsc.get_sparse_core_info()
_NC, _NS = _info.num_cores, _info.num_subcores
_NW = _NC * _NS
_BPW = B // _NW
_NG = _BPW // G


@functools.partial(
    pl.kernel,
    out_type=jax.ShapeDtypeStruct((B, F, D), jnp.float32),
    mesh=plsc.VectorSubcoreMesh(core_axis_name="c", subcore_axis_name="s"),
    scratch_types=[
        pltpu.VMEM((_BPW,), jnp.int32),
        pltpu.VMEM((_BPW,), jnp.int32),
        pltpu.VMEM((G, D), jnp.float32),
        pltpu.VMEM((G, D), jnp.float32),
        pltpu.VMEM((2, L, D), jnp.float32),
        pltpu.VMEM((2, F, D), jnp.float32),
        pltpu.SemaphoreType.DMA,
        pltpu.SemaphoreType.DMA,
        pltpu.SemaphoreType.DMA((2,)),
        pltpu.SemaphoreType.DMA((2,)),
    ],
)
def _sc_all(f_hbm, gidx_hbm, didx_hbm, wg_hbm, wd_hbm, out_hbm,
            gidx_v, didx_v, g8, d8, fbuf, obuf,
            sem_g, sem_d, sem_in, sem_out):
    wid = lax.axis_index("s") * _NC + lax.axis_index("c")
    base = wid * _BPW
    pltpu.sync_copy(gidx_hbm.at[pl.ds(base, _BPW)], gidx_v)
    pltpu.sync_copy(didx_hbm.at[pl.ds(base, _BPW)], didx_v)

    def g_copy(grp):
        return pltpu.make_async_copy(
            wg_hbm.at[gidx_v.at[pl.ds(grp * G, G)]], g8, sem_g)

    def d_copy(grp):
        return pltpu.make_async_copy(
            wd_hbm.at[didx_v.at[pl.ds(grp * G, G)]], d8, sem_d)

    def in_copy(k):
        r = lax.rem(k, 2)
        return pltpu.make_async_copy(f_hbm.at[base + k], fbuf.at[r],
                                     sem_in.at[r])

    def out_copy(k):
        r = lax.rem(k, 2)
        return pltpu.make_async_copy(obuf.at[r], out_hbm.at[base + k],
                                     sem_out.at[r])

    g_copy(0).start()
    d_copy(0).start()
    in_copy(0).start()
    in_copy(1).start()

    def step(k, _):
        r = lax.rem(k, 2)
        grp = lax.div(k, G)
        row = lax.rem(k, G)
        in_copy(k).wait()

        @pl.when(row == 0)
        def _():
            g_copy(grp).wait()
            d_copy(grp).wait()

        @pl.when(k >= 2)
        def _():
            out_copy(k - 2).wait()

        def shift_rows(j, _):
            j2 = 2 * j
            for i in range(D // 16):
                sl = pl.ds(i * 16, 16)
                obuf[r, j2 + 2, sl] = fbuf[r, j2, sl]
                obuf[r, j2 + 3, sl] = fbuf[r, j2 + 1, sl]
            return 0

        lax.fori_loop(0, L // 2, shift_rows, 0)
        for i in range(D // 16):
            sl = pl.ds(i * 16, 16)
            obuf[r, 0, sl] = g8[row, sl]
            obuf[r, 1, sl] = d8[row, sl]
        out_copy(k).start()

        @pl.when(k + 2 < _BPW)
        def _():
            in_copy(k + 2).start()

        @pl.when((row == G - 1) & (grp + 1 < _NG))
        def _():
            g_copy(grp + 1).start()
            d_copy(grp + 1).start()

        return 0

    lax.fori_loop(0, _BPW, step, 0)
    out_copy(_BPW - 2).wait()
    out_copy(_BPW - 1).wait()


def kernel(feature, genre_index, difficulty_index, W_genre, W_difficulty):
    gidx = genre_index.reshape(B).astype(jnp.int32)
    didx = difficulty_index.reshape(B).astype(jnp.int32)
    return _sc_all(feature, gidx, didx, W_genre, W_difficulty)


# speedup vs baseline: 1.5043x; 1.4083x over previous
import functools

import jax
import jax.numpy as jnp
from jax import lax
from jax.experimental import pallas as pl
from jax.experimental.pallas import tpu as pltpu
from jax.experimental.pallas import tpu_sc as plsc

B, L, D = 1024, 50, 512
F = L + 2
_info = plsc.get_sparse_core_info()
_NC, _NS = _info.num_cores, _info.num_subcores
_NW = _NC * _NS
_BPW = B // _NW


@functools.partial(
    pl.kernel,
    out_type=(
        jax.ShapeDtypeStruct((B, D), jnp.float32),
        jax.ShapeDtypeStruct((B, D), jnp.float32),
    ),
    mesh=plsc.VectorSubcoreMesh(core_axis_name="c", subcore_axis_name="s"),
    scratch_types=[
        pltpu.VMEM((_BPW,), jnp.int32),
        pltpu.VMEM((_BPW,), jnp.int32),
        pltpu.VMEM((_BPW, D), jnp.float32),
        pltpu.VMEM((_BPW, D), jnp.float32),
        pltpu.SemaphoreType.DMA,
        pltpu.SemaphoreType.DMA,
    ],
)
def _sc_gather(gidx_hbm, didx_hbm, wg_hbm, wd_hbm, outg_hbm, outd_hbm,
               gidx_v, didx_v, grows_v, drows_v, sem_g, sem_d):
    wid = lax.axis_index("s") * _NC + lax.axis_index("c")
    base = wid * _BPW
    pltpu.sync_copy(gidx_hbm.at[pl.ds(base, _BPW)], gidx_v)
    pltpu.sync_copy(didx_hbm.at[pl.ds(base, _BPW)], didx_v)
    cg = pltpu.async_copy(wg_hbm.at[gidx_v], grows_v, sem_g)
    cd = pltpu.async_copy(wd_hbm.at[didx_v], drows_v, sem_d)
    cg.wait()
    cd.wait()
    pltpu.sync_copy(grows_v, outg_hbm.at[pl.ds(base, _BPW)])
    pltpu.sync_copy(drows_v, outd_hbm.at[pl.ds(base, _BPW)])


_BB = 16
_NB = 4
_C = B // _BB


def _tc_body(f_hbm, g_ref, d_ref, o_hbm,
             fb0, fb1, ob0, ob1, is0, is1, os0, os1):
    fbs, obs, iss, oss = (fb0, fb1), (ob0, ob1), (is0, is1), (os0, os1)

    def in_copy(i):
        q, s = i % 2, (i // 2) % _NB
        return pltpu.make_async_copy(f_hbm.at[pl.ds(i * _BB, _BB)],
                                     fbs[q].at[s], iss[q].at[s])

    def out_copy(i):
        q, s = i % 2, (i // 2) % _NB
        return pltpu.make_async_copy(obs[q].at[s],
                                     o_hbm.at[pl.ds(i * _BB, _BB)],
                                     oss[q].at[s])

    for i in range(2 * _NB):
        in_copy(i).start()
    for i in range(_C):
        in_copy(i).wait()
        if i >= 2 * _NB:
            out_copy(i - 2 * _NB).wait()
        q, s = i % 2, (i // 2) % _NB
        obs[q][s, :, 2:, :] = fbs[q][s]
        obs[q][s, :, 0, :] = g_ref[pl.ds(i * _BB, _BB), :]
        obs[q][s, :, 1, :] = d_ref[pl.ds(i * _BB, _BB), :]
        out_copy(i).start()
        if i + 2 * _NB < _C:
            in_copy(i + 2 * _NB).start()
    for i in range(_C - 2 * _NB, _C):
        out_copy(i).wait()


def _tc_assemble(feature, embg, embd):
    return pl.pallas_call(
        _tc_body,
        in_specs=[
            pl.BlockSpec(memory_space=pl.ANY),
            pl.BlockSpec((B, D), lambda: (0, 0)),
            pl.BlockSpec((B, D), lambda: (0, 0)),
        ],
        out_specs=pl.BlockSpec(memory_space=pl.ANY),
        out_shape=jax.ShapeDtypeStruct((B, F, D), jnp.float32),
        scratch_shapes=[
            pltpu.VMEM((_NB, _BB, L, D), jnp.float32),
            pltpu.VMEM((_NB, _BB, L, D), jnp.float32),
            pltpu.VMEM((_NB, _BB, F, D), jnp.float32),
            pltpu.VMEM((_NB, _BB, F, D), jnp.float32),
            pltpu.SemaphoreType.DMA((_NB,)),
            pltpu.SemaphoreType.DMA((_NB,)),
            pltpu.SemaphoreType.DMA((_NB,)),
            pltpu.SemaphoreType.DMA((_NB,)),
        ],
    )(feature, embg, embd)


def kernel(feature, genre_index, difficulty_index, W_genre, W_difficulty):
    gidx = genre_index.reshape(B).astype(jnp.int32)
    didx = difficulty_index.reshape(B).astype(jnp.int32)
    embg, embd = _sc_gather(gidx, didx, W_genre, W_difficulty)
    return _tc_assemble(feature, embg, embd)
